# baseline (speedup 1.0000x reference)
import functools

import jax
import jax.numpy as jnp
from jax import lax
from jax.experimental import pallas as pl
from jax.experimental.pallas import tpu as pltpu
from jax.experimental.pallas import tpu_sc as plsc

CHUNK = 128
NBUF = 2
NC, NS, L = 2, 16, 16
NW = NC * NS

_mesh = plsc.VectorSubcoreMesh(core_axis_name="c", subcore_axis_name="s")


def _make_pairify(v: int, d: int):
    w = 2 * d
    units = v // w
    tail = v - units * w
    per_w = units // NW
    extra = units - per_w * NW
    n_steps = (per_w + NBUF) // NBUF

    @functools.partial(
        pl.kernel,
        mesh=_mesh,
        compiler_params=pltpu.CompilerParams(needs_layout_passes=False),
        out_type=jax.ShapeDtypeStruct((v // 2, w), jnp.float32),
        scratch_types=[
            pltpu.VMEM((NBUF, d, w), jnp.float32),
            pltpu.VMEM((NBUF, d, w), jnp.float32),
            pltpu.SemaphoreType.DMA,
            pltpu.SemaphoreType.DMA,
        ],
    )
    def _pairify(tt_hbm, tail_hbm, tp_hbm, s_v, d_v, sem0, sem1):
        wid = lax.axis_index("s") * NC + lax.axis_index("c")
        nu = per_w + jnp.where(wid < extra, 1, 0)
        u0 = wid * per_w + jnp.minimum(wid, extra)
        sems = [sem0, sem1]
        iota = lax.iota(jnp.int32, L)

        def istart(t):
            return pl.multiple_of((u0 + t) * w, w)

        def fire(t, buf):
            for di in range(d // 8):
                pltpu.async_copy(
                    tt_hbm.at[pl.ds(di * 8, 8), pl.ds(istart(t), w)],
                    s_v.at[buf].at[pl.ds(di * 8, 8)], sems[buf],
                )

        def drain(buf):
            pltpu.make_async_copy(
                tt_hbm.at[:, pl.ds(0, w)], s_v.at[buf], sems[buf]
            ).wait()

        fire(0, 0)

        def step(p, carry):
            for b in range(NBUF):
                u = p * NBUF + b

                @pl.when(u + 1 < nu)
                def _():
                    fire(u + 1, (b + 1) % NBUF)

                @pl.when(u < nu)
                def _():
                    drain(b)

                    def prow(p2, carry2):
                        for j16 in range(w // L):
                            rvec = (j16 % (d // L)) * L + iota
                            col = 2 * p2 + (1 if j16 >= d // L else 0)
                            vv = plsc.load_gather(
                                s_v.at[b],
                                [rvec, jnp.full((L,), 0, jnp.int32) + col],
                            )
                            d_v[b, p2, pl.ds(j16 * L, L)] = vv
                        return carry2

                    lax.fori_loop(0, d, prow, 0, unroll=4)
                    pltpu.sync_copy(
                        d_v.at[b],
                        tp_hbm.at[pl.ds(pl.multiple_of(
                            lax.shift_right_logical(istart(u), 1), d), d)],
                    )
            return carry

        lax.fori_loop(0, n_steps, step, 0)

        if tail:
            @pl.when(wid == NW - 1)
            def _():
                pltpu.sync_copy(
                    tail_hbm, d_v.at[0].at[pl.ds(0, tail // 2)]
                )
                pltpu.sync_copy(
                    d_v.at[0].at[pl.ds(0, tail // 2)],
                    tp_hbm.at[pl.ds(units * w // 2, tail // 2)],
                )

    return _pairify


def _make_gather(n: int, d: int):
    rows_per_w = n // NW
    assert rows_per_w % (CHUNK * NBUF) == 0
    n_chunks = rows_per_w // CHUNK

    @functools.partial(
        pl.kernel,
        mesh=_mesh,
        compiler_params=pltpu.CompilerParams(needs_layout_passes=False),
        out_type=jax.ShapeDtypeStruct((n, d), jnp.float32),
        scratch_types=[
            pltpu.VMEM((rows_per_w,), jnp.int32),
            pltpu.VMEM((NBUF, CHUNK), jnp.int32),
            pltpu.VMEM((NBUF, CHUNK, 2 * d), jnp.float32),
            pltpu.VMEM((NBUF, CHUNK, d), jnp.float32),
            pltpu.SemaphoreType.DMA,
            pltpu.SemaphoreType.DMA,
        ],
    )
    def _gather(idx_hbm, tp_hbm, out_hbm, idx_v, pair_v, pairs_v, out_v,
                sem0, sem1):
        wid = lax.axis_index("s") * NC + lax.axis_index("c")
        base = wid * rows_per_w
        sems = [sem0, sem1]

        pltpu.sync_copy(idx_hbm.at[pl.ds(base, rows_per_w)], idx_v)

        def fire(c, buf):
            for g in range(CHUNK // L):
                iv = idx_v[pl.ds(c * CHUNK + g * L, L)]
                pair_v[buf, pl.ds(g * L, L)] = lax.shift_right_logical(iv, 1)
            pltpu.async_copy(
                tp_hbm.at[pair_v.at[buf]], pairs_v.at[buf], sems[buf]
            )

        def drain(buf):
            pltpu.make_async_copy(
                tp_hbm.at[pl.ds(0, CHUNK)], pairs_v.at[buf], sems[buf]
            ).wait()

        def compact(c, buf):
            def group(g, carry):
                for u in range(L):
                    r = g * L + u
                    hv = plsc.load_gather(
                        idx_v, [jnp.full((L,), c * CHUNK + r, jnp.int32)]
                    )
                    m = lax.bitwise_and(hv, 1) != 0
                    for j4 in range(d // L):
                        lo = pairs_v[buf, r, pl.ds(j4 * L, L)]
                        hi = pairs_v[buf, r, pl.ds(d + j4 * L, L)]
                        out_v[buf, r, pl.ds(j4 * L, L)] = jnp.where(m, hi, lo)
                return carry

            lax.fori_loop(0, CHUNK // L, group, 0)

        fire(0, 0)

        def pair_step(p, carry):
            c0 = p * NBUF
            for b in range(NBUF):
                c = c0 + b

                @pl.when(c + 1 < n_chunks)
                def _():
                    fire(c + 1, (b + 1) % NBUF)

                drain(b)
                compact(c, b)
                pltpu.sync_copy(
                    out_v.at[b], out_hbm.at[pl.ds(base + c * CHUNK, CHUNK)]
                )
            return carry

        lax.fori_loop(0, n_chunks // NBUF, pair_step, 0)

    return _gather


def kernel(x, table):
    b, h = x.shape
    v, d = table.shape
    flat = x.reshape(-1).astype(jnp.int32)
    units = v // (2 * d)
    tail_p = table[units * 2 * d:, :].reshape(-1, 2 * d)
    tp = _make_pairify(v, d)(table.T, tail_p)
    out = _make_gather(flat.shape[0], d)(flat, tp)
    return out.reshape(b, h, d)


# speedup vs baseline: 2.7546x; 2.7546x over previous
import functools

import jax
import jax.numpy as jnp
from jax import lax
from jax.experimental import pallas as pl
from jax.experimental.pallas import tpu as pltpu
from jax.experimental.pallas import tpu_sc as plsc

CHUNK = 128
NBUF = 2
NC, NS, L = 2, 16, 16
NW = NC * NS

_mesh = plsc.VectorSubcoreMesh(core_axis_name="c", subcore_axis_name="s")


def _make_pairify(v: int, d: int):
    w = 2 * d
    units = v // w
    tail = v - units * w
    per_w = units // NW
    extra = units - per_w * NW
    n_steps = (per_w + NBUF) // NBUF

    @functools.partial(
        pl.kernel,
        mesh=_mesh,
        compiler_params=pltpu.CompilerParams(needs_layout_passes=False),
        out_type=jax.ShapeDtypeStruct((v // 2, w), jnp.float32),
        scratch_types=[
            pltpu.VMEM((NBUF, d, w), jnp.float32),
            pltpu.VMEM((NBUF, d, w), jnp.float32),
            pltpu.SemaphoreType.DMA,
            pltpu.SemaphoreType.DMA,
        ],
    )
    def _pairify(tt_hbm, tail_hbm, tp_hbm, s_v, d_v, sem0, sem1):
        wid = lax.axis_index("s") * NC + lax.axis_index("c")
        nu = per_w + jnp.where(wid < extra, 1, 0)
        u0 = wid * per_w + jnp.minimum(wid, extra)
        sems = [sem0, sem1]
        iota = lax.iota(jnp.int32, L)

        def istart(t):
            return pl.multiple_of((u0 + t) * w, w)

        def fire(t, buf):
            for di in range(d // 8):
                pltpu.async_copy(
                    tt_hbm.at[pl.ds(di * 8, 8), pl.ds(istart(t), w)],
                    s_v.at[buf].at[pl.ds(di * 8, 8)], sems[buf],
                )

        def drain(buf):
            pltpu.make_async_copy(
                tt_hbm.at[:, pl.ds(0, w)], s_v.at[buf], sems[buf]
            ).wait()

        fire(0, 0)

        def step(p, carry):
            for b in range(NBUF):
                u = p * NBUF + b

                @pl.when(u + 1 < nu)
                def _():
                    fire(u + 1, (b + 1) % NBUF)

                @pl.when(u < nu)
                def _():
                    drain(b)

                    rvs = []
                    cvbs = []
                    for g in range(w // L):
                        lv = g * L + iota
                        rvs.append(lax.shift_right_logical(lv, 1))
                        cvbs.append(lax.bitwise_and(lv, 1) * d)

                    @functools.partial(plsc.parallel_loop, 0, d, unroll=8)
                    def _row(j):
                        for g in range(w // L):
                            plsc.store_scatter(
                                d_v.at[b], [rvs[g], cvbs[g] + j],
                                s_v[b, j, pl.ds(g * L, L)],
                            )

                    pltpu.sync_copy(
                        d_v.at[b],
                        tp_hbm.at[pl.ds(pl.multiple_of(
                            lax.shift_right_logical(istart(u), 1), d), d)],
                    )
            return carry

        lax.fori_loop(0, n_steps, step, 0)

        if tail:
            @pl.when(wid == NW - 1)
            def _():
                pltpu.sync_copy(
                    tail_hbm, d_v.at[0].at[pl.ds(0, tail // 2)]
                )
                pltpu.sync_copy(
                    d_v.at[0].at[pl.ds(0, tail // 2)],
                    tp_hbm.at[pl.ds(units * w // 2, tail // 2)],
                )

    return _pairify


def _make_gather(n: int, d: int):
    rows_per_w = n // NW
    assert rows_per_w % (CHUNK * NBUF) == 0
    n_chunks = rows_per_w // CHUNK

    @functools.partial(
        pl.kernel,
        mesh=_mesh,
        compiler_params=pltpu.CompilerParams(needs_layout_passes=False),
        out_type=jax.ShapeDtypeStruct((n, d), jnp.float32),
        scratch_types=[
            pltpu.VMEM((rows_per_w,), jnp.int32),
            pltpu.VMEM((NBUF, CHUNK), jnp.int32),
            pltpu.VMEM((NBUF, CHUNK, 2 * d), jnp.float32),
            pltpu.VMEM((NBUF, CHUNK, d), jnp.float32),
            pltpu.SemaphoreType.DMA,
            pltpu.SemaphoreType.DMA,
        ],
    )
    def _gather(idx_hbm, tp_hbm, out_hbm, idx_v, pair_v, pairs_v, out_v,
                sem0, sem1):
        wid = lax.axis_index("s") * NC + lax.axis_index("c")
        base = wid * rows_per_w
        sems = [sem0, sem1]

        pltpu.sync_copy(idx_hbm.at[pl.ds(base, rows_per_w)], idx_v)

        def fire(c, buf):
            for g in range(CHUNK // L):
                iv = idx_v[pl.ds(c * CHUNK + g * L, L)]
                pair_v[buf, pl.ds(g * L, L)] = lax.shift_right_logical(iv, 1)
            pltpu.async_copy(
                tp_hbm.at[pair_v.at[buf]], pairs_v.at[buf], sems[buf]
            )

        def drain(buf):
            pltpu.make_async_copy(
                tp_hbm.at[pl.ds(0, CHUNK)], pairs_v.at[buf], sems[buf]
            ).wait()

        def compact(c, buf):
            def group(g, carry):
                for u in range(L):
                    r = g * L + u
                    hv = plsc.load_gather(
                        idx_v, [jnp.full((L,), c * CHUNK + r, jnp.int32)]
                    )
                    m = lax.bitwise_and(hv, 1) != 0
                    for j4 in range(d // L):
                        lo = pairs_v[buf, r, pl.ds(j4 * L, L)]
                        hi = pairs_v[buf, r, pl.ds(d + j4 * L, L)]
                        out_v[buf, r, pl.ds(j4 * L, L)] = jnp.where(m, hi, lo)
                return carry

            lax.fori_loop(0, CHUNK // L, group, 0)

        fire(0, 0)

        def pair_step(p, carry):
            c0 = p * NBUF
            for b in range(NBUF):
                c = c0 + b

                @pl.when(c + 1 < n_chunks)
                def _():
                    fire(c + 1, (b + 1) % NBUF)

                drain(b)
                compact(c, b)
                pltpu.sync_copy(
                    out_v.at[b], out_hbm.at[pl.ds(base + c * CHUNK, CHUNK)]
                )
            return carry

        lax.fori_loop(0, n_chunks // NBUF, pair_step, 0)

    return _gather


def kernel(x, table):
    b, h = x.shape
    v, d = table.shape
    flat = x.reshape(-1).astype(jnp.int32)
    units = v // (2 * d)
    tail_p = table[units * 2 * d:, :].reshape(-1, 2 * d)
    tp = _make_pairify(v, d)(table.T, tail_p)
    out = _make_gather(flat.shape[0], d)(flat, tp)
    return out.reshape(b, h, d)
